# Initial kernel scaffold; baseline (speedup 1.0000x reference)
import jax
import jax.numpy as jnp
from jax.experimental import pallas as pl


def kernel(features, Wc, bc, Wo, bo, Wl, bl):
    raise NotImplementedError("write your pallas kernel here")


# speedup vs baseline: 15.4070x; 15.4070x over previous
import functools

import jax
import jax.numpy as jnp
from jax.experimental import pallas as pl

_N = 4
_C = 384
_HF = 64
_NPOS = _HF * _HF
_P = 9
_IMG = 512.0
_K1 = 2000
_KOUT = 1000
_OFF_SCALE = 64.0
_NEG = -1e9
_NMS_T = 0.7


def _conv_head_body(x0_ref, x1_ref, x2_ref, w9_ref, bc_ref, wh_ref, bh_ref,
                    g_ref, bx1_ref, by1_ref, bx2_ref, by2_ref, s_ref):
    xrefs = (x0_ref, x1_ref, x2_ref)
    wcat = w9_ref[...].reshape(9 * _C, _C).astype(jnp.bfloat16)
    whead = wh_ref[...].astype(jnp.bfloat16)
    mc = 512
    for m0 in range(0, _NPOS, mc):
        xcat = jnp.concatenate(
            [xrefs[tap % 3][0, (tap // 3) * _HF + m0:
                            (tap // 3) * _HF + m0 + mc, :]
             .astype(jnp.bfloat16) for tap in range(9)], axis=1)
        acc = jnp.dot(xcat, wcat, preferred_element_type=jnp.float32)
        t = jnp.maximum(acc + bc_ref[0][None, :], 0.0)
        head = jnp.dot(t.astype(jnp.bfloat16), whead,
                       preferred_element_type=jnp.float32)
        head = head + bh_ref[0][None, :]
        e = (jnp.exp(head[:, 0:18]) - 1.0) * _OFF_SCALE
        cx = e[:, 0:_P] + g_ref[m0:m0 + mc, 0:1]
        cy = e[:, _P:2 * _P] + g_ref[m0:m0 + mc, 1:2]
        x1 = jnp.clip(jnp.min(cx, axis=1), 0.0, _IMG - 1.0)
        x2 = jnp.clip(jnp.max(cx, axis=1), 0.0, _IMG - 1.0)
        y1 = jnp.clip(jnp.min(cy, axis=1), 0.0, _IMG - 1.0)
        y2 = jnp.clip(jnp.max(cy, axis=1), 0.0, _IMG - 1.0)
        bx1_ref[0, 0, m0:m0 + mc] = x1
        by1_ref[0, 0, m0:m0 + mc] = y1
        bx2_ref[0, 0, m0:m0 + mc] = x2
        by2_ref[0, 0, m0:m0 + mc] = y2
        s_ref[0, 0, m0:m0 + mc] = head[:, 18]


def _nms_body(x1_ref, y1_ref, x2_ref, y2_ref, s_in_ref,
              ox1_ref, oy1_ref, ox2_ref, oy2_ref, os_ref):
    x1 = x1_ref[...]
    y1 = y1_ref[...]
    x2 = x2_ref[...]
    y2 = y2_ref[...]
    areas = (x2 - x1) * (y2 - y1)
    iota = jax.lax.broadcasted_iota(jnp.int32, (_N, _NPOS), 1).astype(jnp.float32)
    s_nat = s_in_ref[...]

    msb = jnp.int32(-2**31)
    u = jax.lax.bitcast_convert_type(s_nat, jnp.int32)
    key = jnp.where(u >= 0, u, jnp.bitwise_xor(jnp.bitwise_not(u), msb))

    def bis_body(_, lohi):
        lo, hi = lohi
        favg = jnp.bitwise_and(lo, hi) + jnp.right_shift(
            jnp.bitwise_xor(lo, hi), 1)
        mid = favg + jnp.bitwise_and(jnp.bitwise_xor(lo, hi), 1)
        cnt = jnp.sum((key >= mid).astype(jnp.int32), axis=1, keepdims=True)
        go = cnt >= _K1
        lo = jnp.where(go, mid, lo)
        hi = jnp.where(go, hi, mid - 1)
        return lo, hi

    lo0 = jnp.full((_N, 1), msb, jnp.int32)
    hi0 = jnp.full((_N, 1), 2**31 - 1, jnp.int32)
    t_key, _ = jax.lax.fori_loop(0, 32, bis_body, (lo0, hi0))
    s0 = jnp.where(key >= t_key, s_nat, _NEG)

    def pick(s):
        mx = jnp.max(s, axis=1, keepdims=True)
        idx = jnp.where(s == mx, iota, 1e9)
        return mx, jnp.min(idx, axis=1, keepdims=True)

    _, m0 = pick(s0)
    oiota = jax.lax.broadcasted_iota(jnp.int32, (_N, _KOUT), 1)

    def body(i, carry):
        s, ox1, oy1, ox2, oy2, osc = carry
        mx, j = pick(s)
        j = jnp.where(mx > _NEG, j, m0)
        eq = iota == j
        sel_x1 = jnp.sum(jnp.where(eq, x1, 0.0), axis=1, keepdims=True)
        sel_y1 = jnp.sum(jnp.where(eq, y1, 0.0), axis=1, keepdims=True)
        sel_x2 = jnp.sum(jnp.where(eq, x2, 0.0), axis=1, keepdims=True)
        sel_y2 = jnp.sum(jnp.where(eq, y2, 0.0), axis=1, keepdims=True)
        slot = oiota == i
        ox1 = jnp.where(slot, sel_x1, ox1)
        oy1 = jnp.where(slot, sel_y1, oy1)
        ox2 = jnp.where(slot, sel_x2, ox2)
        oy2 = jnp.where(slot, sel_y2, oy2)
        osc = jnp.where(slot, mx, osc)
        ix1 = jnp.maximum(sel_x1, x1)
        iy1 = jnp.maximum(sel_y1, y1)
        ix2 = jnp.minimum(sel_x2, x2)
        iy2 = jnp.minimum(sel_y2, y2)
        inter = jnp.maximum(ix2 - ix1, 0.0) * jnp.maximum(iy2 - iy1, 0.0)
        area_b = (sel_x2 - sel_x1) * (sel_y2 - sel_y1)
        iou = inter / (area_b + areas - inter + 1e-8)
        s = jnp.where((iou > _NMS_T) | eq, _NEG, s)
        return s, ox1, oy1, ox2, oy2, osc

    z = jnp.zeros((_N, _KOUT), jnp.float32)
    _, ox1, oy1, ox2, oy2, osc = jax.lax.fori_loop(
        0, _KOUT, body, (s0, z, z, z, z, z))
    ox1_ref[...] = ox1
    oy1_ref[...] = oy1
    ox2_ref[...] = ox2
    oy2_ref[...] = oy2
    os_ref[...] = osc


def _conv_stage(features, Wc, bc, Wo, bo, Wl, bl):
    xt = jnp.transpose(features, (0, 2, 3, 1))
    xp = jnp.pad(xt, ((0, 0), (1, 1), (1, 1), (0, 0)))
    xsh = [xp[:, :, d:d + _HF, :].reshape(_N, 66 * _HF, _C) for d in range(3)]
    w9 = jnp.transpose(Wc, (2, 3, 1, 0)).reshape(9, _C, _C)
    wo2 = Wo[:, :, 0, 0]
    wh_t = jnp.concatenate([wo2[0::2], wo2[1::2], Wl[:, :, 0, 0]], axis=0)
    wh = jnp.zeros((_C, 128), jnp.float32).at[:, 0:19].set(wh_t.T)
    bh_t = jnp.concatenate([bo[0::2], bo[1::2], bl], axis=0)
    bh = jnp.zeros((1, 128), jnp.float32).at[0, 0:19].set(bh_t)
    lin = jnp.linspace(0.0, _IMG - 1.0, _HF)
    gx = jnp.tile(lin, _HF)
    gy = jnp.repeat(lin, _HF)
    g = jnp.zeros((_NPOS, 8), jnp.float32).at[:, 0].set(gx).at[:, 1].set(gy)
    bc2 = bc.reshape(1, _C)

    out5 = [jax.ShapeDtypeStruct((_N, 1, _NPOS), jnp.float32)] * 5
    xspec = pl.BlockSpec((1, 66 * _HF, _C), lambda b: (b, 0, 0))
    full = lambda shape: pl.BlockSpec(shape, lambda b: tuple(0 for _ in shape))
    ospec = pl.BlockSpec((1, 1, _NPOS), lambda b: (b, 0, 0))
    bx1, by1, bx2, by2, s = pl.pallas_call(
        _conv_head_body,
        grid=(_N,),
        in_specs=[xspec, xspec, xspec,
                  full((9, _C, _C)), full((1, _C)), full((_C, 128)),
                  full((1, 128)), full((_NPOS, 8))],
        out_specs=[ospec] * 5,
        out_shape=out5,
    )(xsh[0], xsh[1], xsh[2], w9, bc2, wh, bh, g)
    return tuple(a.reshape(_N, _NPOS) for a in (bx1, by1, bx2, by2, s))


def kernel(features, Wc, bc, Wo, bo, Wl, bl):
    bx1, by1, bx2, by2, s = _conv_stage(features, Wc, bc, Wo, bo, Wl, bl)

    oshape = [jax.ShapeDtypeStruct((_N, _KOUT), jnp.float32)] * 5
    fullspec = pl.BlockSpec((_N, _NPOS), lambda: (0, 0))
    outspec = pl.BlockSpec((_N, _KOUT), lambda: (0, 0))
    ox1, oy1, ox2, oy2, osc = pl.pallas_call(
        _nms_body,
        grid=(),
        in_specs=[fullspec] * 5,
        out_specs=[outspec] * 5,
        out_shape=oshape,
    )(bx1, by1, bx2, by2, s)

    boxes = jnp.stack([ox1, oy1, ox2, oy2], axis=-1)
    return boxes, osc
